# Initial kernel scaffold; baseline (speedup 1.0000x reference)
import jax
import jax.numpy as jnp
from jax.experimental import pallas as pl


def kernel(node_feature, node_type, edge_time, edge_index, edge_type, adapt_W, adapt_b, Wk, bk, Wq, bq, Wv, bv, Wa, ba, rel_pri, rel_att, rel_msg, skip, ln_g, ln_b, rte_W, rte_b, rte_emb):
    raise NotImplementedError("write your pallas kernel here")


# speedup vs baseline: 2.2056x; 2.2056x over previous
import functools
import math

import jax
import jax.numpy as jnp
import numpy as np
from jax import lax
from jax.experimental import pallas as pl
from jax.experimental.pallas import tpu as pltpu
from jax.experimental.pallas import tpu_sc as plsc

N = 10000
E = 320000
NH = 128
T = 3
RR = 3
H = 8
DK = 16
L = 2
MAXLEN = 240

NC = 2
NS = 16
NW = NC * NS
EPW = E // NW
CH = 80
NCHUNK = EPW // CH
ROWS_PT = N // NS

NB = 400
NGRID = N // NB
EB = 512
EGRID = E // EB

_ONES16 = np.zeros((NH, 16), np.float32)
for _j in range(NH):
    _ONES16[_j, _j // DK] = 1.0
_REP = np.zeros((16, NH), np.float32)
for _h in range(H):
    _REP[_h, _h * DK:(_h + 1) * DK] = 1.0


def _adapt_body(nf_ref, mask_ref, w_ref, b_ref, o_ref):
    acc = jnp.zeros((NB, NH), jnp.float32)
    for t in range(T):
        z = jnp.dot(nf_ref[...], w_ref[t], preferred_element_type=jnp.float32)
        z = jnp.tanh(z + b_ref[t:t + 1, :])
        acc = acc + mask_ref[:, t:t + 1] * z
    o_ref[...] = acc


def _adapt(nf, mask_nt, w, b):
    return pl.pallas_call(
        _adapt_body,
        grid=(NGRID,),
        in_specs=[
            pl.BlockSpec((NB, NH), lambda i: (i, 0)),
            pl.BlockSpec((NB, T), lambda i: (i, 0)),
            pl.BlockSpec((T, NH, NH), lambda i: (0, 0, 0)),
            pl.BlockSpec((T, NH), lambda i: (0, 0)),
        ],
        out_specs=pl.BlockSpec((NB, NH), lambda i: (i, 0)),
        out_shape=jax.ShapeDtypeStruct((N, NH), jnp.float32),
    )(nf, mask_nt, w, b)


def _prep_body(x_ref, mask_ref, wk_ref, bk_ref, wq_ref, bq_ref, wv_ref, bv_ref,
               br_ref, bm_ref, remb_ref, rw_ref, rb_ref,
               qn_ref, ka_ref, vm_ref, rka_ref, rvm_ref):
    i = pl.program_id(0)
    x = x_ref[...]
    kk = jnp.zeros((NB, NH), jnp.float32)
    vv = jnp.zeros((NB, NH), jnp.float32)
    qq = jnp.zeros((NB, NH), jnp.float32)
    for t in range(T):
        m = mask_ref[:, t:t + 1]
        kk = kk + m * (jnp.dot(x, wk_ref[t], preferred_element_type=jnp.float32) + bk_ref[t:t + 1, :])
        vv = vv + m * (jnp.dot(x, wv_ref[t], preferred_element_type=jnp.float32) + bv_ref[t:t + 1, :])
        qq = qq + m * (jnp.dot(x, wq_ref[t], preferred_element_type=jnp.float32) + bq_ref[t:t + 1, :])
    qn_ref[...] = qq
    for r in range(RR):
        ka_ref[:, r * NH:(r + 1) * NH] = jnp.dot(kk, br_ref[r], preferred_element_type=jnp.float32)
        vm_ref[:, r * NH:(r + 1) * NH] = jnp.dot(vv, bm_ref[r], preferred_element_type=jnp.float32)

    @pl.when(i == 0)
    def _():
        rte = jnp.dot(remb_ref[...], rw_ref[...], preferred_element_type=jnp.float32) + rb_ref[0:1, :]
        for t in range(T):
            rkt = jnp.dot(rte, wk_ref[t], preferred_element_type=jnp.float32)
            rvt = jnp.dot(rte, wv_ref[t], preferred_element_type=jnp.float32)
            for r in range(RR):
                row = (t * RR + r) * MAXLEN
                rka_ref[row:row + MAXLEN, :] = jnp.dot(rkt, br_ref[r], preferred_element_type=jnp.float32)
                rvm_ref[row:row + MAXLEN, :] = jnp.dot(rvt, bm_ref[r], preferred_element_type=jnp.float32)


def _prep(x, mask_nt, wk, bk, wq, bq, wv, bv, br, bm, remb, rw, rb):
    full3 = pl.BlockSpec((T, NH, NH), lambda i: (0, 0, 0))
    full2 = pl.BlockSpec((T, NH), lambda i: (0, 0))
    return pl.pallas_call(
        _prep_body,
        grid=(NGRID,),
        in_specs=[
            pl.BlockSpec((NB, NH), lambda i: (i, 0)),
            pl.BlockSpec((NB, T), lambda i: (i, 0)),
            full3, full2, full3, full2, full3, full2,
            pl.BlockSpec((RR, NH, NH), lambda i: (0, 0, 0)),
            pl.BlockSpec((RR, NH, NH), lambda i: (0, 0, 0)),
            pl.BlockSpec((MAXLEN, 2 * NH), lambda i: (0, 0)),
            pl.BlockSpec((2 * NH, NH), lambda i: (0, 0)),
            pl.BlockSpec((1, NH), lambda i: (0, 0)),
        ],
        out_specs=[
            pl.BlockSpec((NB, NH), lambda i: (i, 0)),
            pl.BlockSpec((NB, RR * NH), lambda i: (i, 0)),
            pl.BlockSpec((NB, RR * NH), lambda i: (i, 0)),
            pl.BlockSpec((T * RR * MAXLEN, NH), lambda i: (0, 0)),
            pl.BlockSpec((T * RR * MAXLEN, NH), lambda i: (0, 0)),
        ],
        out_shape=[
            jax.ShapeDtypeStruct((N, NH), jnp.float32),
            jax.ShapeDtypeStruct((N, RR * NH), jnp.float32),
            jax.ShapeDtypeStruct((N, RR * NH), jnp.float32),
            jax.ShapeDtypeStruct((T * RR * MAXLEN, NH), jnp.float32),
            jax.ShapeDtypeStruct((T * RR * MAXLEN, NH), jnp.float32),
        ],
    )(x, mask_nt, wk, bk, wq, bq, wv, bv, br, bm, remb, rw, rb)


def _logits_body(ke_ref, qe_ref, ones_ref, o_ref):
    o_ref[...] = jnp.dot(ke_ref[...] * qe_ref[...], ones_ref[...],
                         preferred_element_type=jnp.float32)


def _logits(ke, qe, ones16):
    return pl.pallas_call(
        _logits_body,
        grid=(EGRID,),
        in_specs=[
            pl.BlockSpec((EB, NH), lambda i: (i, 0)),
            pl.BlockSpec((EB, NH), lambda i: (i, 0)),
            pl.BlockSpec((NH, 16), lambda i: (0, 0)),
        ],
        out_specs=pl.BlockSpec((EB, 16), lambda i: (i, 0)),
        out_shape=jax.ShapeDtypeStruct((E, 16), jnp.float32),
    )(ke, qe, ones16)


def _gmax_body(lg_ref, o_ref):
    i = pl.program_id(0)

    @pl.when(i == 0)
    def _():
        o_ref[...] = jnp.full((8, NH), -1e30, jnp.float32)

    o_ref[...] = jnp.maximum(o_ref[...], jnp.max(lg_ref[...]))


def _gmax(lg):
    return pl.pallas_call(
        _gmax_body,
        grid=(E // 2000,),
        in_specs=[pl.BlockSpec((2000, 16), lambda i: (i, 0))],
        out_specs=pl.BlockSpec((8, NH), lambda i: (0, 0)),
        out_shape=jax.ShapeDtypeStruct((8, NH), jnp.float32),
    )(lg)


def _epilogue_body(ag_ref, s_ref, x_ref, mask_ref, wa_ref, ba_ref, al_ref,
                   lng_ref, lnb_ref, rep_ref, o_ref):
    a = ag_ref[0] + ag_ref[1]
    ss = s_ref[0] + s_ref[1]
    srep = jnp.dot(ss, rep_ref[...], preferred_element_type=jnp.float32)
    z = a / (srep + 1e-16)
    g = 0.5 * z * (1.0 + lax.erf(z * (1.0 / math.sqrt(2.0))))
    x = x_ref[...]
    out = jnp.zeros((NB, NH), jnp.float32)
    for t in range(T):
        trans = jnp.dot(g, wa_ref[t], preferred_element_type=jnp.float32) + ba_ref[t:t + 1, :]
        al = al_ref[t:t + 1, :]
        hh = trans * al + x * (1.0 - al)
        mu = jnp.mean(hh, axis=-1, keepdims=True)
        dd = hh - mu
        var = jnp.mean(dd * dd, axis=-1, keepdims=True)
        hh = dd * lax.rsqrt(var + 1e-5) * lng_ref[t:t + 1, :] + lnb_ref[t:t + 1, :]
        out = out + mask_ref[:, t:t + 1] * hh
    o_ref[...] = out


def _epilogue(aggr2, s2, x, mask_nt, wa, ba, alphav, lng, lnb, rep):
    return pl.pallas_call(
        _epilogue_body,
        grid=(NGRID,),
        in_specs=[
            pl.BlockSpec((NC, NB, NH), lambda i: (0, i, 0)),
            pl.BlockSpec((NC, NB, 16), lambda i: (0, i, 0)),
            pl.BlockSpec((NB, NH), lambda i: (i, 0)),
            pl.BlockSpec((NB, T), lambda i: (i, 0)),
            pl.BlockSpec((T, NH, NH), lambda i: (0, 0, 0)),
            pl.BlockSpec((T, NH), lambda i: (0, 0)),
            pl.BlockSpec((T, NH), lambda i: (0, 0)),
            pl.BlockSpec((T, NH), lambda i: (0, 0)),
            pl.BlockSpec((T, NH), lambda i: (0, 0)),
            pl.BlockSpec((16, NH), lambda i: (0, 0)),
        ],
        out_specs=pl.BlockSpec((NB, NH), lambda i: (i, 0)),
        out_shape=jax.ShapeDtypeStruct((N, NH), jnp.float32),
    )(aggr2, s2, x, mask_nt, wa, ba, alphav, lng, lnb, rep)


_MESH = plsc.VectorSubcoreMesh(core_axis_name="c", subcore_axis_name="s")


def _sc_gather_body(ia_hbm, ib_hbm, dst_hbm,
                    ka_hbm, rka_hbm, vm_hbm, rvm_hbm, qn_hbm,
                    ke_hbm, ve_hbm, qe_hbm,
                    iav, ibv, dstv,
                    av, bv_, a2v, b2v, qv, sem):
    c = lax.axis_index("c")
    s = lax.axis_index("s")
    wid = s * NC + c
    base = wid * EPW

    def chunk(ch, carry):
        off = base + ch * CH
        pltpu.sync_copy(ia_hbm.at[pl.ds(off, CH)], iav)
        pltpu.sync_copy(ib_hbm.at[pl.ds(off, CH)], ibv)
        pltpu.sync_copy(dst_hbm.at[pl.ds(off, CH)], dstv)

        pltpu.async_copy(ka_hbm.at[iav], av, sem).wait()
        pltpu.async_copy(rka_hbm.at[ibv], bv_, sem).wait()
        pltpu.async_copy(vm_hbm.at[iav], a2v, sem).wait()
        pltpu.async_copy(rvm_hbm.at[ibv], b2v, sem).wait()
        pltpu.async_copy(qn_hbm.at[dstv], qv, sem).wait()

        def add_body(e, carry2):
            for h in range(H):
                sl = pl.ds(h * DK, DK)
                av[e, sl] = av[e, sl] + bv_[e, sl]
                a2v[e, sl] = a2v[e, sl] + b2v[e, sl]
            return carry2

        lax.fori_loop(0, CH, add_body, 0)

        pltpu.sync_copy(av, ke_hbm.at[pl.ds(off, CH)])
        pltpu.sync_copy(a2v, ve_hbm.at[pl.ds(off, CH)])
        pltpu.sync_copy(qv, qe_hbm.at[pl.ds(off, CH)])
        return carry

    lax.fori_loop(0, NCHUNK, chunk, 0)


@functools.partial(
    pl.kernel,
    mesh=_MESH,
    compiler_params=pltpu.CompilerParams(use_tc_tiling_on_sc=False),
    out_type=[
        jax.ShapeDtypeStruct((E, NH), jnp.float32),
        jax.ShapeDtypeStruct((E, NH), jnp.float32),
        jax.ShapeDtypeStruct((E, NH), jnp.float32),
    ],
    scratch_types=[
        pltpu.VMEM((CH,), jnp.int32),
        pltpu.VMEM((CH,), jnp.int32),
        pltpu.VMEM((CH,), jnp.int32),
        pltpu.VMEM((CH, NH), jnp.float32),
        pltpu.VMEM((CH, NH), jnp.float32),
        pltpu.VMEM((CH, NH), jnp.float32),
        pltpu.VMEM((CH, NH), jnp.float32),
        pltpu.VMEM((CH, NH), jnp.float32),
        pltpu.SemaphoreType.DMA,
    ],
)
def _sc_gather(*refs):
    _sc_gather_body(*refs)


def _sc_scatter_body(m_hbm, dst_hbm, lg_hbm, ve_hbm,
                     aggr_hbm, s_hbm,
                     mv, dstv, lgv, vev, msgv, exv, zbuf, zsbuf,
                     aggr_sp, s_sp):
    c = lax.axis_index("c")
    s = lax.axis_index("s")
    wid = s * NC + c
    base = wid * EPW

    zero16 = jnp.zeros((16,), jnp.float32)

    def z1(r, carry):
        for h in range(H):
            zbuf[r, pl.ds(h * DK, DK)] = zero16
        return carry

    lax.fori_loop(0, 64, z1, 0)

    def z2(r, carry):
        zsbuf[r, pl.ds(0, 16)] = zero16
        return carry

    lax.fori_loop(0, 64, z2, 0)

    for k in range(10):
        pltpu.sync_copy(zbuf, aggr_sp.at[pl.ds(s * 624 + k * 64, 64)])
        pltpu.sync_copy(zsbuf, s_sp.at[pl.ds(s * 624 + k * 64, 64)])
    plsc.subcore_barrier()

    pltpu.sync_copy(m_hbm, mv)
    mvec = mv[...]
    lanes = lax.iota(jnp.int32, 16)
    maskv = jnp.where(lanes < H, 1.0, 0.0).astype(jnp.float32)

    def chunk(ch, carry):
        off = base + ch * CH
        pltpu.sync_copy(dst_hbm.at[pl.ds(off, CH)], dstv)
        pltpu.sync_copy(lg_hbm.at[pl.ds(off, CH)], lgv)
        pltpu.sync_copy(ve_hbm.at[pl.ds(off, CH)], vev)

        def e_body(e, carry2):
            lvec = lgv[e]
            ex = jnp.exp(lvec - mvec) * maskv
            exv[e] = ex
            for h in range(H):
                sl = pl.ds(h * DK, DK)
                msgv[e, sl] = vev[e, sl] * ex[h]
            return carry2

        lax.fori_loop(0, CH, e_body, 0)

        pltpu.sync_copy(msgv, aggr_sp.at[dstv], add=True)
        pltpu.sync_copy(exv, s_sp.at[dstv], add=True)
        return carry

    lax.fori_loop(0, NCHUNK, chunk, 0)
    plsc.subcore_barrier()

    for k in range(10):
        r0 = s * 624 + k * 64
        pltpu.sync_copy(aggr_sp.at[pl.ds(r0, 64)], zbuf)
        pltpu.sync_copy(zbuf, aggr_hbm.at[c, pl.ds(r0, 64)])
        pltpu.sync_copy(s_sp.at[pl.ds(r0, 64)], zsbuf)
        pltpu.sync_copy(zsbuf, s_hbm.at[c, pl.ds(r0, 64)])


@functools.partial(
    pl.kernel,
    mesh=_MESH,
    compiler_params=pltpu.CompilerParams(use_tc_tiling_on_sc=False),
    out_type=[
        jax.ShapeDtypeStruct((NC, N, NH), jnp.float32),
        jax.ShapeDtypeStruct((NC, N, 16), jnp.float32),
    ],
    scratch_types=[
        pltpu.VMEM((16,), jnp.float32),
        pltpu.VMEM((CH,), jnp.int32),
        pltpu.VMEM((CH, 16), jnp.float32),
        pltpu.VMEM((CH, NH), jnp.float32),
        pltpu.VMEM((CH, NH), jnp.float32),
        pltpu.VMEM((CH, 16), jnp.float32),
        pltpu.VMEM((64, NH), jnp.float32),
        pltpu.VMEM((64, 16), jnp.float32),
        pltpu.VMEM_SHARED((N, NH), jnp.float32),
        pltpu.VMEM_SHARED((N, 16), jnp.float32),
    ],
)
def _sc_scatter(*refs):
    _sc_scatter_body(*refs)


def kernel(node_feature, node_type, edge_time, edge_index, edge_type,
           adapt_W, adapt_b, Wk, bk, Wq, bq, Wv, bv, Wa, ba,
           rel_pri, rel_att, rel_msg, skip, ln_g, ln_b, rte_W, rte_b, rte_emb):
    node_type = node_type.astype(jnp.int32)
    edge_time = edge_time.astype(jnp.int32)
    edge_type = edge_type.astype(jnp.int32)
    src = edge_index[0].astype(jnp.int32)
    dst = edge_index[1].astype(jnp.int32)

    mask_nt = (node_type[:, None] == jnp.arange(T, dtype=jnp.int32)[None, :]).astype(jnp.float32)
    ones16 = jnp.asarray(_ONES16)
    rep = jnp.asarray(_REP)

    x = _adapt(node_feature, mask_nt, adapt_W, adapt_b)

    for l in range(L):
        scale = (rel_pri[l] / math.sqrt(DK))[:, :, None, None]
        ratt_s = rel_att[l] * scale
        br = jnp.zeros((RR, NH, NH), jnp.float32)
        bm = jnp.zeros((RR, NH, NH), jnp.float32)
        for h in range(H):
            sl = slice(h * DK, (h + 1) * DK)
            br = br.at[:, sl, sl].set(ratt_s[:, h])
            bm = bm.at[:, sl, sl].set(rel_msg[l, :, h])

        qn, kaw, vmw, rka, rvm = _prep(
            x, mask_nt, Wk[l], bk[l], Wq[l], bq[l], Wv[l], bv[l],
            br, bm, rte_emb, rte_W[l], rte_b[l].reshape(1, NH))
        ka = kaw.reshape(N * RR, NH)
        vm = vmw.reshape(N * RR, NH)

        ia = src * RR + edge_type
        ib = (node_type[src] * RR + edge_type) * MAXLEN + edge_time
        ke, ve, qe = _sc_gather(ia, ib, dst, ka, rka, vm, rvm, qn)
        lg = _logits(ke, qe, ones16)
        mx = _gmax(lg)
        mvec = jnp.broadcast_to(mx[0, 0], (16,)).astype(jnp.float32)

        aggr2, s2 = _sc_scatter(mvec, dst, lg, ve)

        alphav = jnp.broadcast_to(jax.nn.sigmoid(skip[l])[:, None], (T, NH))
        x = _epilogue(aggr2, s2, x, mask_nt, Wa[l], ba[l], alphav,
                      ln_g[l], ln_b[l], rep)
    return x


# speedup vs baseline: 2.5969x; 1.1774x over previous
import functools
import math

import jax
import jax.numpy as jnp
import numpy as np
from jax import lax
from jax.experimental import pallas as pl
from jax.experimental.pallas import tpu as pltpu
from jax.experimental.pallas import tpu_sc as plsc

N = 10000
E = 320000
NH = 128
T = 3
RR = 3
H = 8
DK = 16
L = 2
MAXLEN = 240

NC = 2
NS = 16
NW = NC * NS
EPW = E // NW
CH = 80
NCHUNK = EPW // CH
ROWS_PT = N // NS

NB = 400
NGRID = N // NB
EB = 512
EGRID = E // EB

_ONES16 = np.zeros((NH, 16), np.float32)
for _j in range(NH):
    _ONES16[_j, _j // DK] = 1.0
_REP = np.zeros((16, NH), np.float32)
for _h in range(H):
    _REP[_h, _h * DK:(_h + 1) * DK] = 1.0


def _adapt_body(nf_ref, mask_ref, w_ref, b_ref, o_ref):
    acc = jnp.zeros((NB, NH), jnp.float32)
    for t in range(T):
        z = jnp.dot(nf_ref[...], w_ref[t], preferred_element_type=jnp.float32)
        z = jnp.tanh(z + b_ref[t:t + 1, :])
        acc = acc + mask_ref[:, t:t + 1] * z
    o_ref[...] = acc


def _adapt(nf, mask_nt, w, b):
    return pl.pallas_call(
        _adapt_body,
        grid=(NGRID,),
        in_specs=[
            pl.BlockSpec((NB, NH), lambda i: (i, 0)),
            pl.BlockSpec((NB, T), lambda i: (i, 0)),
            pl.BlockSpec((T, NH, NH), lambda i: (0, 0, 0)),
            pl.BlockSpec((T, NH), lambda i: (0, 0)),
        ],
        out_specs=pl.BlockSpec((NB, NH), lambda i: (i, 0)),
        out_shape=jax.ShapeDtypeStruct((N, NH), jnp.float32),
    )(nf, mask_nt, w, b)


def _prep_body(x_ref, mask_ref, wk_ref, bk_ref, wq_ref, bq_ref, wv_ref, bv_ref,
               br_ref, bm_ref, remb_ref, rw_ref, rb_ref,
               qn_ref, ka_ref, vm_ref, rka_ref, rvm_ref):
    i = pl.program_id(0)
    x = x_ref[...]
    kk = jnp.zeros((NB, NH), jnp.float32)
    vv = jnp.zeros((NB, NH), jnp.float32)
    qq = jnp.zeros((NB, NH), jnp.float32)
    for t in range(T):
        m = mask_ref[:, t:t + 1]
        kk = kk + m * (jnp.dot(x, wk_ref[t], preferred_element_type=jnp.float32) + bk_ref[t:t + 1, :])
        vv = vv + m * (jnp.dot(x, wv_ref[t], preferred_element_type=jnp.float32) + bv_ref[t:t + 1, :])
        qq = qq + m * (jnp.dot(x, wq_ref[t], preferred_element_type=jnp.float32) + bq_ref[t:t + 1, :])
    qn_ref[...] = qq
    for r in range(RR):
        ka_ref[:, r * NH:(r + 1) * NH] = jnp.dot(kk, br_ref[r], preferred_element_type=jnp.float32)
        vm_ref[:, r * NH:(r + 1) * NH] = jnp.dot(vv, bm_ref[r], preferred_element_type=jnp.float32)

    @pl.when(i == 0)
    def _():
        rte = jnp.dot(remb_ref[...], rw_ref[...], preferred_element_type=jnp.float32) + rb_ref[0:1, :]
        for t in range(T):
            rkt = jnp.dot(rte, wk_ref[t], preferred_element_type=jnp.float32)
            rvt = jnp.dot(rte, wv_ref[t], preferred_element_type=jnp.float32)
            for r in range(RR):
                row = (t * RR + r) * MAXLEN
                rka_ref[row:row + MAXLEN, :] = jnp.dot(rkt, br_ref[r], preferred_element_type=jnp.float32)
                rvm_ref[row:row + MAXLEN, :] = jnp.dot(rvt, bm_ref[r], preferred_element_type=jnp.float32)


def _prep(x, mask_nt, wk, bk, wq, bq, wv, bv, br, bm, remb, rw, rb):
    full3 = pl.BlockSpec((T, NH, NH), lambda i: (0, 0, 0))
    full2 = pl.BlockSpec((T, NH), lambda i: (0, 0))
    return pl.pallas_call(
        _prep_body,
        grid=(NGRID,),
        in_specs=[
            pl.BlockSpec((NB, NH), lambda i: (i, 0)),
            pl.BlockSpec((NB, T), lambda i: (i, 0)),
            full3, full2, full3, full2, full3, full2,
            pl.BlockSpec((RR, NH, NH), lambda i: (0, 0, 0)),
            pl.BlockSpec((RR, NH, NH), lambda i: (0, 0, 0)),
            pl.BlockSpec((MAXLEN, 2 * NH), lambda i: (0, 0)),
            pl.BlockSpec((2 * NH, NH), lambda i: (0, 0)),
            pl.BlockSpec((1, NH), lambda i: (0, 0)),
        ],
        out_specs=[
            pl.BlockSpec((NB, NH), lambda i: (i, 0)),
            pl.BlockSpec((NB, RR * NH), lambda i: (i, 0)),
            pl.BlockSpec((NB, RR * NH), lambda i: (i, 0)),
            pl.BlockSpec((T * RR * MAXLEN, NH), lambda i: (0, 0)),
            pl.BlockSpec((T * RR * MAXLEN, NH), lambda i: (0, 0)),
        ],
        out_shape=[
            jax.ShapeDtypeStruct((N, NH), jnp.float32),
            jax.ShapeDtypeStruct((N, RR * NH), jnp.float32),
            jax.ShapeDtypeStruct((N, RR * NH), jnp.float32),
            jax.ShapeDtypeStruct((T * RR * MAXLEN, NH), jnp.float32),
            jax.ShapeDtypeStruct((T * RR * MAXLEN, NH), jnp.float32),
        ],
    )(x, mask_nt, wk, bk, wq, bq, wv, bv, br, bm, remb, rw, rb)


def _logits_body(ke_ref, qe_ref, ones_ref, o_ref, mx_ref):
    i = pl.program_id(0)

    @pl.when(i == 0)
    def _():
        mx_ref[...] = jnp.full((8, NH), -1e30, jnp.float32)

    lg = jnp.dot(ke_ref[...] * qe_ref[...], ones_ref[...],
                 preferred_element_type=jnp.float32)
    o_ref[...] = lg
    mx_ref[...] = jnp.maximum(mx_ref[...], jnp.max(lg))


def _logits(ke, qe, ones16):
    return pl.pallas_call(
        _logits_body,
        grid=(EGRID,),
        in_specs=[
            pl.BlockSpec((EB, NH), lambda i: (i, 0)),
            pl.BlockSpec((EB, NH), lambda i: (i, 0)),
            pl.BlockSpec((NH, 16), lambda i: (0, 0)),
        ],
        out_specs=[
            pl.BlockSpec((EB, 16), lambda i: (i, 0)),
            pl.BlockSpec((8, NH), lambda i: (0, 0)),
        ],
        out_shape=[
            jax.ShapeDtypeStruct((E, 16), jnp.float32),
            jax.ShapeDtypeStruct((8, NH), jnp.float32),
        ],
    )(ke, qe, ones16)


def _epilogue_body(ag_ref, s_ref, x_ref, mask_ref, wa_ref, ba_ref, al_ref,
                   lng_ref, lnb_ref, rep_ref, o_ref):
    a = ag_ref[0] + ag_ref[1]
    ss = s_ref[0] + s_ref[1]
    srep = jnp.dot(ss, rep_ref[...], preferred_element_type=jnp.float32)
    z = a / (srep + 1e-16)
    g = 0.5 * z * (1.0 + lax.erf(z * (1.0 / math.sqrt(2.0))))
    x = x_ref[...]
    out = jnp.zeros((NB, NH), jnp.float32)
    for t in range(T):
        trans = jnp.dot(g, wa_ref[t], preferred_element_type=jnp.float32) + ba_ref[t:t + 1, :]
        al = al_ref[t:t + 1, :]
        hh = trans * al + x * (1.0 - al)
        mu = jnp.mean(hh, axis=-1, keepdims=True)
        dd = hh - mu
        var = jnp.mean(dd * dd, axis=-1, keepdims=True)
        hh = dd * lax.rsqrt(var + 1e-5) * lng_ref[t:t + 1, :] + lnb_ref[t:t + 1, :]
        out = out + mask_ref[:, t:t + 1] * hh
    o_ref[...] = out


def _epilogue(aggr2, s2, x, mask_nt, wa, ba, alphav, lng, lnb, rep):
    return pl.pallas_call(
        _epilogue_body,
        grid=(NGRID,),
        in_specs=[
            pl.BlockSpec((NC, NB, NH), lambda i: (0, i, 0)),
            pl.BlockSpec((NC, NB, 16), lambda i: (0, i, 0)),
            pl.BlockSpec((NB, NH), lambda i: (i, 0)),
            pl.BlockSpec((NB, T), lambda i: (i, 0)),
            pl.BlockSpec((T, NH, NH), lambda i: (0, 0, 0)),
            pl.BlockSpec((T, NH), lambda i: (0, 0)),
            pl.BlockSpec((T, NH), lambda i: (0, 0)),
            pl.BlockSpec((T, NH), lambda i: (0, 0)),
            pl.BlockSpec((T, NH), lambda i: (0, 0)),
            pl.BlockSpec((16, NH), lambda i: (0, 0)),
        ],
        out_specs=pl.BlockSpec((NB, NH), lambda i: (i, 0)),
        out_shape=jax.ShapeDtypeStruct((N, NH), jnp.float32),
    )(aggr2, s2, x, mask_nt, wa, ba, alphav, lng, lnb, rep)


_MESH = plsc.VectorSubcoreMesh(core_axis_name="c", subcore_axis_name="s")


def _sc_gather_body(ia_hbm, ib_hbm, dst_hbm,
                    ka_hbm, rka_hbm, qn_hbm,
                    ke_hbm, qe_hbm,
                    iav, ibv, dstv,
                    av, bv_, qv, sem):
    c = lax.axis_index("c")
    s = lax.axis_index("s")
    wid = s * NC + c
    base = wid * EPW

    def chunk(ch, carry):
        off = base + ch * CH
        pltpu.sync_copy(ia_hbm.at[pl.ds(off, CH)], iav)
        pltpu.sync_copy(ib_hbm.at[pl.ds(off, CH)], ibv)
        pltpu.sync_copy(dst_hbm.at[pl.ds(off, CH)], dstv)

        d1 = pltpu.async_copy(ka_hbm.at[iav], av, sem)
        d2 = pltpu.async_copy(rka_hbm.at[ibv], bv_, sem)
        d3 = pltpu.async_copy(qn_hbm.at[dstv], qv, sem)
        d1.wait()
        d2.wait()
        d3.wait()

        def add_body(e, carry2):
            for h in range(H):
                sl = pl.ds(h * DK, DK)
                av[e, sl] = av[e, sl] + bv_[e, sl]
            return carry2

        lax.fori_loop(0, CH, add_body, 0)

        pltpu.sync_copy(av, ke_hbm.at[pl.ds(off, CH)])
        pltpu.sync_copy(qv, qe_hbm.at[pl.ds(off, CH)])
        return carry

    lax.fori_loop(0, NCHUNK, chunk, 0)


@functools.partial(
    pl.kernel,
    mesh=_MESH,
    compiler_params=pltpu.CompilerParams(use_tc_tiling_on_sc=False),
    out_type=[
        jax.ShapeDtypeStruct((E, NH), jnp.float32),
        jax.ShapeDtypeStruct((E, NH), jnp.float32),
    ],
    scratch_types=[
        pltpu.VMEM((CH,), jnp.int32),
        pltpu.VMEM((CH,), jnp.int32),
        pltpu.VMEM((CH,), jnp.int32),
        pltpu.VMEM((CH, NH), jnp.float32),
        pltpu.VMEM((CH, NH), jnp.float32),
        pltpu.VMEM((CH, NH), jnp.float32),
        pltpu.SemaphoreType.DMA,
    ],
)
def _sc_gather(*refs):
    _sc_gather_body(*refs)


def _sc_scatter_body(m_hbm, dst_hbm, lg_hbm, ia_hbm, ib_hbm, vm_hbm, rvm_hbm,
                     aggr_hbm, s_hbm,
                     mv, dstv, iav, ibv, lgv, a2v, b2v, exv, zbuf, zsbuf, sem,
                     aggr_sp, s_sp):
    c = lax.axis_index("c")
    s = lax.axis_index("s")
    wid = s * NC + c
    base = wid * EPW

    zero16 = jnp.zeros((16,), jnp.float32)

    def z1(r, carry):
        for h in range(H):
            zbuf[r, pl.ds(h * DK, DK)] = zero16
        return carry

    lax.fori_loop(0, 64, z1, 0)

    def z2(r, carry):
        zsbuf[r, pl.ds(0, 16)] = zero16
        return carry

    lax.fori_loop(0, 64, z2, 0)

    for k in range(10):
        pltpu.sync_copy(zbuf, aggr_sp.at[pl.ds(s * 624 + k * 64, 64)])
        pltpu.sync_copy(zsbuf, s_sp.at[pl.ds(s * 624 + k * 64, 64)])
    plsc.subcore_barrier()

    pltpu.sync_copy(m_hbm, mv)
    mvec = mv[...]
    lanes = lax.iota(jnp.int32, 16)
    maskv = jnp.where(lanes < H, 1.0, 0.0).astype(jnp.float32)

    def chunk(ch, carry):
        off = base + ch * CH
        pltpu.sync_copy(ia_hbm.at[pl.ds(off, CH)], iav)
        pltpu.sync_copy(ib_hbm.at[pl.ds(off, CH)], ibv)
        d1 = pltpu.async_copy(vm_hbm.at[iav], a2v, sem)
        d2 = pltpu.async_copy(rvm_hbm.at[ibv], b2v, sem)
        pltpu.sync_copy(dst_hbm.at[pl.ds(off, CH)], dstv)
        pltpu.sync_copy(lg_hbm.at[pl.ds(off, CH)], lgv)
        d1.wait()
        d2.wait()

        def e_body(e, carry2):
            lvec = lgv[e]
            ex = jnp.exp(lvec - mvec) * maskv
            exv[e] = ex
            for h in range(H):
                sl = pl.ds(h * DK, DK)
                a2v[e, sl] = (a2v[e, sl] + b2v[e, sl]) * ex[h]
            return carry2

        lax.fori_loop(0, CH, e_body, 0)

        pltpu.sync_copy(a2v, aggr_sp.at[dstv], add=True)
        pltpu.sync_copy(exv, s_sp.at[dstv], add=True)
        return carry

    lax.fori_loop(0, NCHUNK, chunk, 0)
    plsc.subcore_barrier()

    for k in range(10):
        r0 = s * 624 + k * 64
        pltpu.sync_copy(aggr_sp.at[pl.ds(r0, 64)], zbuf)
        pltpu.sync_copy(zbuf, aggr_hbm.at[c, pl.ds(r0, 64)])
        pltpu.sync_copy(s_sp.at[pl.ds(r0, 64)], zsbuf)
        pltpu.sync_copy(zsbuf, s_hbm.at[c, pl.ds(r0, 64)])


@functools.partial(
    pl.kernel,
    mesh=_MESH,
    compiler_params=pltpu.CompilerParams(use_tc_tiling_on_sc=False),
    out_type=[
        jax.ShapeDtypeStruct((NC, N, NH), jnp.float32),
        jax.ShapeDtypeStruct((NC, N, 16), jnp.float32),
    ],
    scratch_types=[
        pltpu.VMEM((16,), jnp.float32),
        pltpu.VMEM((CH,), jnp.int32),
        pltpu.VMEM((CH,), jnp.int32),
        pltpu.VMEM((CH,), jnp.int32),
        pltpu.VMEM((CH, 16), jnp.float32),
        pltpu.VMEM((CH, NH), jnp.float32),
        pltpu.VMEM((CH, NH), jnp.float32),
        pltpu.VMEM((CH, 16), jnp.float32),
        pltpu.VMEM((64, NH), jnp.float32),
        pltpu.VMEM((64, 16), jnp.float32),
        pltpu.SemaphoreType.DMA,
        pltpu.VMEM_SHARED((N, NH), jnp.float32),
        pltpu.VMEM_SHARED((N, 16), jnp.float32),
    ],
)
def _sc_scatter(*refs):
    _sc_scatter_body(*refs)


def kernel(node_feature, node_type, edge_time, edge_index, edge_type,
           adapt_W, adapt_b, Wk, bk, Wq, bq, Wv, bv, Wa, ba,
           rel_pri, rel_att, rel_msg, skip, ln_g, ln_b, rte_W, rte_b, rte_emb):
    node_type = node_type.astype(jnp.int32)
    edge_time = edge_time.astype(jnp.int32)
    edge_type = edge_type.astype(jnp.int32)
    src = edge_index[0].astype(jnp.int32)
    dst = edge_index[1].astype(jnp.int32)

    mask_nt = (node_type[:, None] == jnp.arange(T, dtype=jnp.int32)[None, :]).astype(jnp.float32)
    ones16 = jnp.asarray(_ONES16)
    rep = jnp.asarray(_REP)

    x = _adapt(node_feature, mask_nt, adapt_W, adapt_b)

    for l in range(L):
        scale = (rel_pri[l] / math.sqrt(DK))[:, :, None, None]
        ratt_s = rel_att[l] * scale
        br = jnp.zeros((RR, NH, NH), jnp.float32)
        bm = jnp.zeros((RR, NH, NH), jnp.float32)
        for h in range(H):
            sl = slice(h * DK, (h + 1) * DK)
            br = br.at[:, sl, sl].set(ratt_s[:, h])
            bm = bm.at[:, sl, sl].set(rel_msg[l, :, h])

        qn, kaw, vmw, rka, rvm = _prep(
            x, mask_nt, Wk[l], bk[l], Wq[l], bq[l], Wv[l], bv[l],
            br, bm, rte_emb, rte_W[l], rte_b[l].reshape(1, NH))
        ka = kaw.reshape(N * RR, NH)
        vm = vmw.reshape(N * RR, NH)

        ia = src * RR + edge_type
        ib = (node_type[src] * RR + edge_type) * MAXLEN + edge_time
        ke, qe = _sc_gather(ia, ib, dst, ka, rka, qn)
        lg, mx = _logits(ke, qe, ones16)
        mvec = jnp.broadcast_to(mx[0, 0], (16,)).astype(jnp.float32)

        aggr2, s2 = _sc_scatter(mvec, dst, lg, ia, ib, vm, rvm)

        alphav = jnp.broadcast_to(jax.nn.sigmoid(skip[l])[:, None], (T, NH))
        x = _epilogue(aggr2, s2, x, mask_nt, Wa[l], ba[l], alphav,
                      ln_g[l], ln_b[l], rep)
    return x


# speedup vs baseline: 2.8464x; 1.0961x over previous
import functools
import math

import jax
import jax.numpy as jnp
import numpy as np
from jax import lax
from jax.experimental import pallas as pl
from jax.experimental.pallas import tpu as pltpu
from jax.experimental.pallas import tpu_sc as plsc

N = 10000
E = 320000
NH = 128
T = 3
RR = 3
H = 8
DK = 16
L = 2
MAXLEN = 240

NC = 2
NS = 16
NW = NC * NS
EPW = E // NW
CH = 80
NCHUNK = EPW // CH
ROWS_PT = N // NS

NB = 400
NGRID = N // NB
EB = 2000
EGRID = E // EB

_ONES16 = np.zeros((NH, 16), np.float32)
for _j in range(NH):
    _ONES16[_j, _j // DK] = 1.0
_REP = np.zeros((16, NH), np.float32)
for _h in range(H):
    _REP[_h, _h * DK:(_h + 1) * DK] = 1.0


def _adapt_body(nf_ref, mask_ref, w_ref, b_ref, o_ref):
    acc = jnp.zeros((NB, NH), jnp.float32)
    for t in range(T):
        z = jnp.dot(nf_ref[...], w_ref[t], preferred_element_type=jnp.float32)
        z = jnp.tanh(z + b_ref[t:t + 1, :])
        acc = acc + mask_ref[:, t:t + 1] * z
    o_ref[...] = acc


def _adapt(nf, mask_nt, w, b):
    return pl.pallas_call(
        _adapt_body,
        grid=(NGRID,),
        in_specs=[
            pl.BlockSpec((NB, NH), lambda i: (i, 0)),
            pl.BlockSpec((NB, T), lambda i: (i, 0)),
            pl.BlockSpec((T, NH, NH), lambda i: (0, 0, 0)),
            pl.BlockSpec((T, NH), lambda i: (0, 0)),
        ],
        out_specs=pl.BlockSpec((NB, NH), lambda i: (i, 0)),
        out_shape=jax.ShapeDtypeStruct((N, NH), jnp.float32),
    )(nf, mask_nt, w, b)


def _prep_body(x_ref, mask_ref, wk_ref, bk_ref, wq_ref, bq_ref, wv_ref, bv_ref,
               br_ref, bm_ref, remb_ref, rw_ref, rb_ref,
               qn_ref, ka_ref, vm_ref, rka_ref, rvm_ref):
    i = pl.program_id(0)
    x = x_ref[...]
    kk = jnp.zeros((NB, NH), jnp.float32)
    vv = jnp.zeros((NB, NH), jnp.float32)
    qq = jnp.zeros((NB, NH), jnp.float32)
    for t in range(T):
        m = mask_ref[:, t:t + 1]
        kk = kk + m * (jnp.dot(x, wk_ref[t], preferred_element_type=jnp.float32) + bk_ref[t:t + 1, :])
        vv = vv + m * (jnp.dot(x, wv_ref[t], preferred_element_type=jnp.float32) + bv_ref[t:t + 1, :])
        qq = qq + m * (jnp.dot(x, wq_ref[t], preferred_element_type=jnp.float32) + bq_ref[t:t + 1, :])
    qn_ref[...] = qq
    for r in range(RR):
        ka_ref[:, r * NH:(r + 1) * NH] = jnp.dot(kk, br_ref[r], preferred_element_type=jnp.float32)
        vm_ref[:, r * NH:(r + 1) * NH] = jnp.dot(vv, bm_ref[r], preferred_element_type=jnp.float32)

    @pl.when(i == 0)
    def _():
        rte = jnp.dot(remb_ref[...], rw_ref[...], preferred_element_type=jnp.float32) + rb_ref[0:1, :]
        for t in range(T):
            rkt = jnp.dot(rte, wk_ref[t], preferred_element_type=jnp.float32)
            rvt = jnp.dot(rte, wv_ref[t], preferred_element_type=jnp.float32)
            for r in range(RR):
                row = (t * RR + r) * MAXLEN
                rka_ref[row:row + MAXLEN, :] = jnp.dot(rkt, br_ref[r], preferred_element_type=jnp.float32)
                rvm_ref[row:row + MAXLEN, :] = jnp.dot(rvt, bm_ref[r], preferred_element_type=jnp.float32)


def _prep(x, mask_nt, wk, bk, wq, bq, wv, bv, br, bm, remb, rw, rb):
    full3 = pl.BlockSpec((T, NH, NH), lambda i: (0, 0, 0))
    full2 = pl.BlockSpec((T, NH), lambda i: (0, 0))
    return pl.pallas_call(
        _prep_body,
        grid=(NGRID,),
        in_specs=[
            pl.BlockSpec((NB, NH), lambda i: (i, 0)),
            pl.BlockSpec((NB, T), lambda i: (i, 0)),
            full3, full2, full3, full2, full3, full2,
            pl.BlockSpec((RR, NH, NH), lambda i: (0, 0, 0)),
            pl.BlockSpec((RR, NH, NH), lambda i: (0, 0, 0)),
            pl.BlockSpec((MAXLEN, 2 * NH), lambda i: (0, 0)),
            pl.BlockSpec((2 * NH, NH), lambda i: (0, 0)),
            pl.BlockSpec((1, NH), lambda i: (0, 0)),
        ],
        out_specs=[
            pl.BlockSpec((NB, NH), lambda i: (i, 0)),
            pl.BlockSpec((NB, RR * NH), lambda i: (i, 0)),
            pl.BlockSpec((NB, RR * NH), lambda i: (i, 0)),
            pl.BlockSpec((T * RR * MAXLEN, NH), lambda i: (0, 0)),
            pl.BlockSpec((T * RR * MAXLEN, NH), lambda i: (0, 0)),
        ],
        out_shape=[
            jax.ShapeDtypeStruct((N, NH), jnp.float32),
            jax.ShapeDtypeStruct((N, RR * NH), jnp.float32),
            jax.ShapeDtypeStruct((N, RR * NH), jnp.float32),
            jax.ShapeDtypeStruct((T * RR * MAXLEN, NH), jnp.float32),
            jax.ShapeDtypeStruct((T * RR * MAXLEN, NH), jnp.float32),
        ],
    )(x, mask_nt, wk, bk, wq, bq, wv, bv, br, bm, remb, rw, rb)


def _logits_body(ke_ref, qe_ref, ones_ref, o_ref, mx_ref):
    i = pl.program_id(0)

    @pl.when(i == 0)
    def _():
        mx_ref[...] = jnp.full((8, NH), -1e30, jnp.float32)

    lg = jnp.dot(ke_ref[...] * qe_ref[...], ones_ref[...],
                 preferred_element_type=jnp.float32)
    o_ref[...] = lg
    mx_ref[...] = jnp.maximum(mx_ref[...], jnp.max(lg))


def _logits(ke, qe, ones16):
    return pl.pallas_call(
        _logits_body,
        grid=(EGRID,),
        in_specs=[
            pl.BlockSpec((EB, NH), lambda i: (i, 0)),
            pl.BlockSpec((EB, NH), lambda i: (i, 0)),
            pl.BlockSpec((NH, 16), lambda i: (0, 0)),
        ],
        out_specs=[
            pl.BlockSpec((EB, 16), lambda i: (i, 0)),
            pl.BlockSpec((8, NH), lambda i: (0, 0)),
        ],
        out_shape=[
            jax.ShapeDtypeStruct((E, 16), jnp.float32),
            jax.ShapeDtypeStruct((8, NH), jnp.float32),
        ],
    )(ke, qe, ones16)


def _epilogue_body(ag_ref, s_ref, x_ref, mask_ref, wa_ref, ba_ref, al_ref,
                   lng_ref, lnb_ref, rep_ref, o_ref):
    a = ag_ref[0] + ag_ref[1]
    ss = s_ref[0] + s_ref[1]
    srep = jnp.dot(ss, rep_ref[...], preferred_element_type=jnp.float32)
    z = a / (srep + 1e-16)
    g = 0.5 * z * (1.0 + lax.erf(z * (1.0 / math.sqrt(2.0))))
    x = x_ref[...]
    out = jnp.zeros((NB, NH), jnp.float32)
    for t in range(T):
        trans = jnp.dot(g, wa_ref[t], preferred_element_type=jnp.float32) + ba_ref[t:t + 1, :]
        al = al_ref[t:t + 1, :]
        hh = trans * al + x * (1.0 - al)
        mu = jnp.mean(hh, axis=-1, keepdims=True)
        dd = hh - mu
        var = jnp.mean(dd * dd, axis=-1, keepdims=True)
        hh = dd * lax.rsqrt(var + 1e-5) * lng_ref[t:t + 1, :] + lnb_ref[t:t + 1, :]
        out = out + mask_ref[:, t:t + 1] * hh
    o_ref[...] = out


def _epilogue(aggr2, s2, x, mask_nt, wa, ba, alphav, lng, lnb, rep):
    return pl.pallas_call(
        _epilogue_body,
        grid=(NGRID,),
        in_specs=[
            pl.BlockSpec((NC, NB, NH), lambda i: (0, i, 0)),
            pl.BlockSpec((NC, NB, 16), lambda i: (0, i, 0)),
            pl.BlockSpec((NB, NH), lambda i: (i, 0)),
            pl.BlockSpec((NB, T), lambda i: (i, 0)),
            pl.BlockSpec((T, NH, NH), lambda i: (0, 0, 0)),
            pl.BlockSpec((T, NH), lambda i: (0, 0)),
            pl.BlockSpec((T, NH), lambda i: (0, 0)),
            pl.BlockSpec((T, NH), lambda i: (0, 0)),
            pl.BlockSpec((T, NH), lambda i: (0, 0)),
            pl.BlockSpec((16, NH), lambda i: (0, 0)),
        ],
        out_specs=pl.BlockSpec((NB, NH), lambda i: (i, 0)),
        out_shape=jax.ShapeDtypeStruct((N, NH), jnp.float32),
    )(aggr2, s2, x, mask_nt, wa, ba, alphav, lng, lnb, rep)


_MESH = plsc.VectorSubcoreMesh(core_axis_name="c", subcore_axis_name="s")


def _sc_gather_body(ia_hbm, ib_hbm, dst_hbm,
                    ka_hbm, rka_hbm, qn_hbm,
                    ke_hbm, qe_hbm,
                    iav, ibv, dstv,
                    av, bv_, qv, sem):
    c = lax.axis_index("c")
    s = lax.axis_index("s")
    wid = s * NC + c
    base = wid * EPW

    def chunk(ch, carry):
        off = base + ch * CH
        pltpu.sync_copy(ia_hbm.at[pl.ds(off, CH)], iav)
        pltpu.sync_copy(ib_hbm.at[pl.ds(off, CH)], ibv)
        pltpu.sync_copy(dst_hbm.at[pl.ds(off, CH)], dstv)

        d1 = pltpu.async_copy(ka_hbm.at[iav], av, sem)
        d2 = pltpu.async_copy(rka_hbm.at[ibv], bv_, sem)
        d3 = pltpu.async_copy(qn_hbm.at[dstv], qv, sem)
        d1.wait()
        d2.wait()
        d3.wait()

        def add_body(e, carry2):
            for h in range(H):
                sl = pl.ds(h * DK, DK)
                av[e, sl] = av[e, sl] + bv_[e, sl]
            return carry2

        lax.fori_loop(0, CH, add_body, 0)

        pltpu.sync_copy(av, ke_hbm.at[pl.ds(off, CH)])
        pltpu.sync_copy(qv, qe_hbm.at[pl.ds(off, CH)])
        return carry

    lax.fori_loop(0, NCHUNK, chunk, 0)


@functools.partial(
    pl.kernel,
    mesh=_MESH,
    compiler_params=pltpu.CompilerParams(use_tc_tiling_on_sc=False),
    out_type=[
        jax.ShapeDtypeStruct((E, NH), jnp.float32),
        jax.ShapeDtypeStruct((E, NH), jnp.float32),
    ],
    scratch_types=[
        pltpu.VMEM((CH,), jnp.int32),
        pltpu.VMEM((CH,), jnp.int32),
        pltpu.VMEM((CH,), jnp.int32),
        pltpu.VMEM((CH, NH), jnp.float32),
        pltpu.VMEM((CH, NH), jnp.float32),
        pltpu.VMEM((CH, NH), jnp.float32),
        pltpu.SemaphoreType.DMA,
    ],
)
def _sc_gather(*refs):
    _sc_gather_body(*refs)


def _sc_scatter_body(m_hbm, dst_hbm, lg_hbm, ia_hbm, ib_hbm, vm_hbm, rvm_hbm,
                     aggr_hbm, s_hbm,
                     mv, dstv, iav, ibv, lgv, a2v, b2v, exv, zbuf, zsbuf, sem,
                     aggr_sp, s_sp):
    c = lax.axis_index("c")
    s = lax.axis_index("s")
    wid = s * NC + c
    base = wid * EPW

    zero16 = jnp.zeros((16,), jnp.float32)

    def z1(r, carry):
        for h in range(H):
            zbuf[r, pl.ds(h * DK, DK)] = zero16
        return carry

    lax.fori_loop(0, 64, z1, 0)

    def z2(r, carry):
        zsbuf[r, pl.ds(0, 16)] = zero16
        return carry

    lax.fori_loop(0, 64, z2, 0)

    for k in range(10):
        pltpu.sync_copy(zbuf, aggr_sp.at[pl.ds(s * 624 + k * 64, 64)])
        pltpu.sync_copy(zsbuf, s_sp.at[pl.ds(s * 624 + k * 64, 64)])
    plsc.subcore_barrier()

    pltpu.sync_copy(m_hbm, mv)
    mvec = mv[...]
    lanes = lax.iota(jnp.int32, 16)
    maskv = jnp.where(lanes < H, 1.0, 0.0).astype(jnp.float32)

    def chunk(ch, carry):
        off = base + ch * CH
        pltpu.sync_copy(ia_hbm.at[pl.ds(off, CH)], iav)
        pltpu.sync_copy(ib_hbm.at[pl.ds(off, CH)], ibv)
        d1 = pltpu.async_copy(vm_hbm.at[iav], a2v, sem)
        d2 = pltpu.async_copy(rvm_hbm.at[ibv], b2v, sem)
        pltpu.sync_copy(dst_hbm.at[pl.ds(off, CH)], dstv)
        pltpu.sync_copy(lg_hbm.at[pl.ds(off, CH)], lgv)
        d1.wait()
        d2.wait()

        def e_body(e, carry2):
            lvec = lgv[e]
            ex = jnp.exp(lvec - mvec) * maskv
            exv[e] = ex
            for h in range(H):
                sl = pl.ds(h * DK, DK)
                a2v[e, sl] = (a2v[e, sl] + b2v[e, sl]) * ex[h]
            return carry2

        lax.fori_loop(0, CH, e_body, 0)

        pltpu.sync_copy(a2v, aggr_sp.at[dstv], add=True)
        pltpu.sync_copy(exv, s_sp.at[dstv], add=True)
        return carry

    lax.fori_loop(0, NCHUNK, chunk, 0)
    plsc.subcore_barrier()

    for k in range(10):
        r0 = s * 624 + k * 64
        pltpu.sync_copy(aggr_sp.at[pl.ds(r0, 64)], zbuf)
        pltpu.sync_copy(zbuf, aggr_hbm.at[c, pl.ds(r0, 64)])
        pltpu.sync_copy(s_sp.at[pl.ds(r0, 64)], zsbuf)
        pltpu.sync_copy(zsbuf, s_hbm.at[c, pl.ds(r0, 64)])


@functools.partial(
    pl.kernel,
    mesh=_MESH,
    compiler_params=pltpu.CompilerParams(use_tc_tiling_on_sc=False),
    out_type=[
        jax.ShapeDtypeStruct((NC, N, NH), jnp.float32),
        jax.ShapeDtypeStruct((NC, N, 16), jnp.float32),
    ],
    scratch_types=[
        pltpu.VMEM((16,), jnp.float32),
        pltpu.VMEM((CH,), jnp.int32),
        pltpu.VMEM((CH,), jnp.int32),
        pltpu.VMEM((CH,), jnp.int32),
        pltpu.VMEM((CH, 16), jnp.float32),
        pltpu.VMEM((CH, NH), jnp.float32),
        pltpu.VMEM((CH, NH), jnp.float32),
        pltpu.VMEM((CH, 16), jnp.float32),
        pltpu.VMEM((64, NH), jnp.float32),
        pltpu.VMEM((64, 16), jnp.float32),
        pltpu.SemaphoreType.DMA,
        pltpu.VMEM_SHARED((N, NH), jnp.float32),
        pltpu.VMEM_SHARED((N, 16), jnp.float32),
    ],
)
def _sc_scatter(*refs):
    _sc_scatter_body(*refs)


def kernel(node_feature, node_type, edge_time, edge_index, edge_type,
           adapt_W, adapt_b, Wk, bk, Wq, bq, Wv, bv, Wa, ba,
           rel_pri, rel_att, rel_msg, skip, ln_g, ln_b, rte_W, rte_b, rte_emb):
    node_type = node_type.astype(jnp.int32)
    edge_time = edge_time.astype(jnp.int32)
    edge_type = edge_type.astype(jnp.int32)
    src = edge_index[0].astype(jnp.int32)
    dst = edge_index[1].astype(jnp.int32)

    mask_nt = (node_type[:, None] == jnp.arange(T, dtype=jnp.int32)[None, :]).astype(jnp.float32)
    ones16 = jnp.asarray(_ONES16)
    rep = jnp.asarray(_REP)

    x = _adapt(node_feature, mask_nt, adapt_W, adapt_b)

    ia = src * RR + edge_type
    ib = (node_type[src] * RR + edge_type) * MAXLEN + edge_time

    for l in range(L):
        scale = (rel_pri[l] / math.sqrt(DK))[:, :, None, None]
        ratt_s = rel_att[l] * scale
        br = jnp.zeros((RR, NH, NH), jnp.float32)
        bm = jnp.zeros((RR, NH, NH), jnp.float32)
        for h in range(H):
            sl = slice(h * DK, (h + 1) * DK)
            br = br.at[:, sl, sl].set(ratt_s[:, h])
            bm = bm.at[:, sl, sl].set(rel_msg[l, :, h])

        qn, kaw, vmw, rka, rvm = _prep(
            x, mask_nt, Wk[l], bk[l], Wq[l], bq[l], Wv[l], bv[l],
            br, bm, rte_emb, rte_W[l], rte_b[l].reshape(1, NH))
        ka = kaw.reshape(N * RR, NH)
        vm = vmw.reshape(N * RR, NH)

        ke, qe = _sc_gather(ia, ib, dst, ka, rka, qn)
        lg, mx = _logits(ke, qe, ones16)
        mvec = jnp.broadcast_to(mx[0, 0], (16,)).astype(jnp.float32)

        aggr2, s2 = _sc_scatter(mvec, dst, lg, ia, ib, vm, rvm)

        alphav = jnp.broadcast_to(jax.nn.sigmoid(skip[l])[:, None], (T, NH))
        x = _epilogue(aggr2, s2, x, mask_nt, Wa[l], ba[l], alphav,
                      ln_g[l], ln_b[l], rep)
    return x


# speedup vs baseline: 2.9902x; 1.0505x over previous
import functools
import math

import jax
import jax.numpy as jnp
import numpy as np
from jax import lax
from jax.experimental import pallas as pl
from jax.experimental.pallas import tpu as pltpu
from jax.experimental.pallas import tpu_sc as plsc

N = 10000
E = 320000
NH = 128
T = 3
RR = 3
H = 8
DK = 16
L = 2
MAXLEN = 240

NC = 2
NS = 16
NW = NC * NS
EPW = E // NW
CH = 80
NCHUNK = EPW // CH
ROWS_PT = N // NS

NB = 400
NGRID = N // NB
EB = 2000
EGRID = E // EB

_ONES16 = np.zeros((NH, 16), np.float32)
for _j in range(NH):
    _ONES16[_j, _j // DK] = 1.0
_REP = np.zeros((16, NH), np.float32)
for _h in range(H):
    _REP[_h, _h * DK:(_h + 1) * DK] = 1.0


def _adapt_body(nf_ref, mask_ref, w_ref, b_ref, o_ref):
    acc = jnp.zeros((NB, NH), jnp.float32)
    for t in range(T):
        z = jnp.dot(nf_ref[...], w_ref[t], preferred_element_type=jnp.float32)
        z = jnp.tanh(z + b_ref[t:t + 1, :])
        acc = acc + mask_ref[:, t:t + 1] * z
    o_ref[...] = acc


def _adapt(nf, mask_nt, w, b):
    return pl.pallas_call(
        _adapt_body,
        grid=(NGRID,),
        in_specs=[
            pl.BlockSpec((NB, NH), lambda i: (i, 0)),
            pl.BlockSpec((NB, T), lambda i: (i, 0)),
            pl.BlockSpec((T, NH, NH), lambda i: (0, 0, 0)),
            pl.BlockSpec((T, NH), lambda i: (0, 0)),
        ],
        out_specs=pl.BlockSpec((NB, NH), lambda i: (i, 0)),
        out_shape=jax.ShapeDtypeStruct((N, NH), jnp.float32),
    )(nf, mask_nt, w, b)


def _prep_body(x_ref, mask_ref, wk_ref, bk_ref, wq_ref, bq_ref, wv_ref, bv_ref,
               br_ref, bm_ref, remb_ref, rw_ref, rb_ref,
               qn_ref, ka_ref, vm_ref, rka_ref, rvm_ref):
    i = pl.program_id(0)
    x = x_ref[...]
    kk = jnp.zeros((NB, NH), jnp.float32)
    vv = jnp.zeros((NB, NH), jnp.float32)
    qq = jnp.zeros((NB, NH), jnp.float32)
    for t in range(T):
        m = mask_ref[:, t:t + 1]
        kk = kk + m * (jnp.dot(x, wk_ref[t], preferred_element_type=jnp.float32) + bk_ref[t:t + 1, :])
        vv = vv + m * (jnp.dot(x, wv_ref[t], preferred_element_type=jnp.float32) + bv_ref[t:t + 1, :])
        qq = qq + m * (jnp.dot(x, wq_ref[t], preferred_element_type=jnp.float32) + bq_ref[t:t + 1, :])
    qn_ref[...] = qq
    for r in range(RR):
        ka_ref[:, r * NH:(r + 1) * NH] = jnp.dot(kk, br_ref[r], preferred_element_type=jnp.float32)
        vm_ref[:, r * NH:(r + 1) * NH] = jnp.dot(vv, bm_ref[r], preferred_element_type=jnp.float32)

    @pl.when(i == 0)
    def _():
        rte = jnp.dot(remb_ref[...], rw_ref[...], preferred_element_type=jnp.float32) + rb_ref[0:1, :]
        for t in range(T):
            rkt = jnp.dot(rte, wk_ref[t], preferred_element_type=jnp.float32)
            rvt = jnp.dot(rte, wv_ref[t], preferred_element_type=jnp.float32)
            for r in range(RR):
                row = (t * RR + r) * MAXLEN
                rka_ref[row:row + MAXLEN, :] = jnp.dot(rkt, br_ref[r], preferred_element_type=jnp.float32)
                rvm_ref[row:row + MAXLEN, :] = jnp.dot(rvt, bm_ref[r], preferred_element_type=jnp.float32)


def _prep(x, mask_nt, wk, bk, wq, bq, wv, bv, br, bm, remb, rw, rb):
    full3 = pl.BlockSpec((T, NH, NH), lambda i: (0, 0, 0))
    full2 = pl.BlockSpec((T, NH), lambda i: (0, 0))
    return pl.pallas_call(
        _prep_body,
        grid=(NGRID,),
        in_specs=[
            pl.BlockSpec((NB, NH), lambda i: (i, 0)),
            pl.BlockSpec((NB, T), lambda i: (i, 0)),
            full3, full2, full3, full2, full3, full2,
            pl.BlockSpec((RR, NH, NH), lambda i: (0, 0, 0)),
            pl.BlockSpec((RR, NH, NH), lambda i: (0, 0, 0)),
            pl.BlockSpec((MAXLEN, 2 * NH), lambda i: (0, 0)),
            pl.BlockSpec((2 * NH, NH), lambda i: (0, 0)),
            pl.BlockSpec((1, NH), lambda i: (0, 0)),
        ],
        out_specs=[
            pl.BlockSpec((NB, NH), lambda i: (i, 0)),
            pl.BlockSpec((NB, RR * NH), lambda i: (i, 0)),
            pl.BlockSpec((NB, RR * NH), lambda i: (i, 0)),
            pl.BlockSpec((T * RR * MAXLEN, NH), lambda i: (0, 0)),
            pl.BlockSpec((T * RR * MAXLEN, NH), lambda i: (0, 0)),
        ],
        out_shape=[
            jax.ShapeDtypeStruct((N, NH), jnp.float32),
            jax.ShapeDtypeStruct((N, RR * NH), jnp.float32),
            jax.ShapeDtypeStruct((N, RR * NH), jnp.float32),
            jax.ShapeDtypeStruct((T * RR * MAXLEN, NH), jnp.float32),
            jax.ShapeDtypeStruct((T * RR * MAXLEN, NH), jnp.float32),
        ],
    )(x, mask_nt, wk, bk, wq, bq, wv, bv, br, bm, remb, rw, rb)


def _logits_body(ke_ref, qe_ref, ones_ref, o_ref, mx_ref):
    i = pl.program_id(0)

    @pl.when(i == 0)
    def _():
        mx_ref[...] = jnp.full((8, NH), -1e30, jnp.float32)

    lg = jnp.dot(ke_ref[...] * qe_ref[...], ones_ref[...],
                 preferred_element_type=jnp.float32)
    o_ref[...] = lg
    mx_ref[...] = jnp.maximum(mx_ref[...], jnp.max(lg))


def _logits(ke, qe, ones16):
    return pl.pallas_call(
        _logits_body,
        grid=(EGRID,),
        in_specs=[
            pl.BlockSpec((EB, NH), lambda i: (i, 0)),
            pl.BlockSpec((EB, NH), lambda i: (i, 0)),
            pl.BlockSpec((NH, 16), lambda i: (0, 0)),
        ],
        out_specs=[
            pl.BlockSpec((EB, 16), lambda i: (i, 0)),
            pl.BlockSpec((8, NH), lambda i: (0, 0)),
        ],
        out_shape=[
            jax.ShapeDtypeStruct((E, 16), jnp.float32),
            jax.ShapeDtypeStruct((8, NH), jnp.float32),
        ],
    )(ke, qe, ones16)


def _epilogue_body(ag_ref, s_ref, x_ref, mask_ref, wa_ref, ba_ref, al_ref,
                   lng_ref, lnb_ref, rep_ref, o_ref):
    a = ag_ref[0] + ag_ref[1]
    ss = s_ref[0] + s_ref[1]
    srep = jnp.dot(ss, rep_ref[...], preferred_element_type=jnp.float32)
    z = a / (srep + 1e-16)
    g = 0.5 * z * (1.0 + lax.erf(z * (1.0 / math.sqrt(2.0))))
    x = x_ref[...]
    out = jnp.zeros((NB, NH), jnp.float32)
    for t in range(T):
        trans = jnp.dot(g, wa_ref[t], preferred_element_type=jnp.float32) + ba_ref[t:t + 1, :]
        al = al_ref[t:t + 1, :]
        hh = trans * al + x * (1.0 - al)
        mu = jnp.mean(hh, axis=-1, keepdims=True)
        dd = hh - mu
        var = jnp.mean(dd * dd, axis=-1, keepdims=True)
        hh = dd * lax.rsqrt(var + 1e-5) * lng_ref[t:t + 1, :] + lnb_ref[t:t + 1, :]
        out = out + mask_ref[:, t:t + 1] * hh
    o_ref[...] = out


def _epilogue(aggr2, s2, x, mask_nt, wa, ba, alphav, lng, lnb, rep):
    return pl.pallas_call(
        _epilogue_body,
        grid=(NGRID,),
        in_specs=[
            pl.BlockSpec((NC, NB, NH), lambda i: (0, i, 0)),
            pl.BlockSpec((NC, NB, 16), lambda i: (0, i, 0)),
            pl.BlockSpec((NB, NH), lambda i: (i, 0)),
            pl.BlockSpec((NB, T), lambda i: (i, 0)),
            pl.BlockSpec((T, NH, NH), lambda i: (0, 0, 0)),
            pl.BlockSpec((T, NH), lambda i: (0, 0)),
            pl.BlockSpec((T, NH), lambda i: (0, 0)),
            pl.BlockSpec((T, NH), lambda i: (0, 0)),
            pl.BlockSpec((T, NH), lambda i: (0, 0)),
            pl.BlockSpec((16, NH), lambda i: (0, 0)),
        ],
        out_specs=pl.BlockSpec((NB, NH), lambda i: (i, 0)),
        out_shape=jax.ShapeDtypeStruct((N, NH), jnp.float32),
    )(aggr2, s2, x, mask_nt, wa, ba, alphav, lng, lnb, rep)


_MESH = plsc.VectorSubcoreMesh(core_axis_name="c", subcore_axis_name="s")


def _sc_gather_body(eidx_hbm,
                    ka_hbm, rka_hbm, qn_hbm,
                    ke_hbm, qe_hbm,
                    ebuf,
                    av, bv_, qv, sem):
    c = lax.axis_index("c")
    s = lax.axis_index("s")
    wid = s * NC + c
    base = wid * EPW

    def chunk(ch, carry):
        off = base + ch * CH
        pltpu.sync_copy(eidx_hbm.at[wid, ch], ebuf)

        d1 = pltpu.async_copy(ka_hbm.at[ebuf.at[0]], av, sem)
        d2 = pltpu.async_copy(rka_hbm.at[ebuf.at[1]], bv_, sem)
        d3 = pltpu.async_copy(qn_hbm.at[ebuf.at[2]], qv, sem)
        d1.wait()
        d2.wait()
        d3.wait()

        def add_body(e, carry2):
            for h in range(H):
                sl = pl.ds(h * DK, DK)
                av[e, sl] = av[e, sl] + bv_[e, sl]
            return carry2

        lax.fori_loop(0, CH, add_body, 0)

        pltpu.sync_copy(av, ke_hbm.at[pl.ds(off, CH)])
        pltpu.sync_copy(qv, qe_hbm.at[pl.ds(off, CH)])
        return carry

    lax.fori_loop(0, NCHUNK, chunk, 0)


@functools.partial(
    pl.kernel,
    mesh=_MESH,
    compiler_params=pltpu.CompilerParams(use_tc_tiling_on_sc=False),
    out_type=[
        jax.ShapeDtypeStruct((E, NH), jnp.float32),
        jax.ShapeDtypeStruct((E, NH), jnp.float32),
    ],
    scratch_types=[
        pltpu.VMEM((3, CH), jnp.int32),
        pltpu.VMEM((CH, NH), jnp.float32),
        pltpu.VMEM((CH, NH), jnp.float32),
        pltpu.VMEM((CH, NH), jnp.float32),
        pltpu.SemaphoreType.DMA,
    ],
)
def _sc_gather(*refs):
    _sc_gather_body(*refs)


def _sc_scatter_body(m_hbm, lg_hbm, eidx_hbm, vm_hbm, rvm_hbm,
                     aggr_hbm, s_hbm,
                     mv, ebuf, lgv, a2v, b2v, exv, zbuf, zsbuf, sem,
                     aggr_sp, s_sp):
    c = lax.axis_index("c")
    s = lax.axis_index("s")
    wid = s * NC + c
    base = wid * EPW

    zero16 = jnp.zeros((16,), jnp.float32)

    def z1(r, carry):
        for h in range(H):
            zbuf[r, pl.ds(h * DK, DK)] = zero16
        return carry

    lax.fori_loop(0, 64, z1, 0)

    def z2(r, carry):
        zsbuf[r, pl.ds(0, 16)] = zero16
        return carry

    lax.fori_loop(0, 64, z2, 0)

    for k in range(10):
        pltpu.sync_copy(zbuf, aggr_sp.at[pl.ds(s * 624 + k * 64, 64)])
        pltpu.sync_copy(zsbuf, s_sp.at[pl.ds(s * 624 + k * 64, 64)])
    plsc.subcore_barrier()

    pltpu.sync_copy(m_hbm, mv)
    mvec = mv[...]
    lanes = lax.iota(jnp.int32, 16)
    maskv = jnp.where(lanes < H, 1.0, 0.0).astype(jnp.float32)

    def chunk(ch, carry):
        off = base + ch * CH
        d0 = pltpu.async_copy(lg_hbm.at[pl.ds(off, CH)], lgv, sem)
        pltpu.sync_copy(eidx_hbm.at[wid, ch], ebuf)
        d1 = pltpu.async_copy(vm_hbm.at[ebuf.at[0]], a2v, sem)
        d2 = pltpu.async_copy(rvm_hbm.at[ebuf.at[1]], b2v, sem)
        d0.wait()
        d1.wait()
        d2.wait()

        def e_body(e, carry2):
            lvec = lgv[e]
            ex = jnp.exp(lvec - mvec) * maskv
            exv[e] = ex
            for h in range(H):
                sl = pl.ds(h * DK, DK)
                a2v[e, sl] = (a2v[e, sl] + b2v[e, sl]) * ex[h]
            return carry2

        lax.fori_loop(0, CH, e_body, 0)

        pltpu.sync_copy(a2v, aggr_sp.at[ebuf.at[2]], add=True)
        pltpu.sync_copy(exv, s_sp.at[ebuf.at[2]], add=True)
        return carry

    lax.fori_loop(0, NCHUNK, chunk, 0)
    plsc.subcore_barrier()

    for k in range(10):
        r0 = s * 624 + k * 64
        pltpu.sync_copy(aggr_sp.at[pl.ds(r0, 64)], zbuf)
        pltpu.sync_copy(zbuf, aggr_hbm.at[c, pl.ds(r0, 64)])
        pltpu.sync_copy(s_sp.at[pl.ds(r0, 64)], zsbuf)
        pltpu.sync_copy(zsbuf, s_hbm.at[c, pl.ds(r0, 64)])


@functools.partial(
    pl.kernel,
    mesh=_MESH,
    compiler_params=pltpu.CompilerParams(use_tc_tiling_on_sc=False),
    out_type=[
        jax.ShapeDtypeStruct((NC, N, NH), jnp.float32),
        jax.ShapeDtypeStruct((NC, N, 16), jnp.float32),
    ],
    scratch_types=[
        pltpu.VMEM((16,), jnp.float32),
        pltpu.VMEM((3, CH), jnp.int32),
        pltpu.VMEM((CH, 16), jnp.float32),
        pltpu.VMEM((CH, NH), jnp.float32),
        pltpu.VMEM((CH, NH), jnp.float32),
        pltpu.VMEM((CH, 16), jnp.float32),
        pltpu.VMEM((64, NH), jnp.float32),
        pltpu.VMEM((64, 16), jnp.float32),
        pltpu.SemaphoreType.DMA,
        pltpu.VMEM_SHARED((N, NH), jnp.float32),
        pltpu.VMEM_SHARED((N, 16), jnp.float32),
    ],
)
def _sc_scatter(*refs):
    _sc_scatter_body(*refs)


def kernel(node_feature, node_type, edge_time, edge_index, edge_type,
           adapt_W, adapt_b, Wk, bk, Wq, bq, Wv, bv, Wa, ba,
           rel_pri, rel_att, rel_msg, skip, ln_g, ln_b, rte_W, rte_b, rte_emb):
    node_type = node_type.astype(jnp.int32)
    edge_time = edge_time.astype(jnp.int32)
    edge_type = edge_type.astype(jnp.int32)
    src = edge_index[0].astype(jnp.int32)
    dst = edge_index[1].astype(jnp.int32)

    mask_nt = (node_type[:, None] == jnp.arange(T, dtype=jnp.int32)[None, :]).astype(jnp.float32)
    ones16 = jnp.asarray(_ONES16)
    rep = jnp.asarray(_REP)

    x = _adapt(node_feature, mask_nt, adapt_W, adapt_b)

    ia = src * RR + edge_type
    ib = (node_type[src] * RR + edge_type) * MAXLEN + edge_time
    eidx = jnp.stack([ia, ib, dst], axis=0)
    eidx = eidx.reshape(3, NW, NCHUNK, CH).transpose(1, 2, 0, 3)

    for l in range(L):
        scale = (rel_pri[l] / math.sqrt(DK))[:, :, None, None]
        ratt_s = rel_att[l] * scale
        br = jnp.zeros((RR, NH, NH), jnp.float32)
        bm = jnp.zeros((RR, NH, NH), jnp.float32)
        for h in range(H):
            sl = slice(h * DK, (h + 1) * DK)
            br = br.at[:, sl, sl].set(ratt_s[:, h])
            bm = bm.at[:, sl, sl].set(rel_msg[l, :, h])

        qn, kaw, vmw, rka, rvm = _prep(
            x, mask_nt, Wk[l], bk[l], Wq[l], bq[l], Wv[l], bv[l],
            br, bm, rte_emb, rte_W[l], rte_b[l].reshape(1, NH))
        ka = kaw.reshape(N * RR, NH)
        vm = vmw.reshape(N * RR, NH)

        ke, qe = _sc_gather(eidx, ka, rka, qn)
        lg, mx = _logits(ke, qe, ones16)
        mvec = jnp.broadcast_to(mx[0, 0], (16,)).astype(jnp.float32)

        aggr2, s2 = _sc_scatter(mvec, lg, eidx, vm, rvm)

        alphav = jnp.broadcast_to(jax.nn.sigmoid(skip[l])[:, None], (T, NH))
        x = _epilogue(aggr2, s2, x, mask_nt, Wa[l], ba[l], alphav,
                      ln_g[l], ln_b[l], rep)
    return x
